# baseline (speedup 1.0000x reference)
import functools
import jax
import jax.numpy as jnp
from jax import lax
from jax.experimental import pallas as pl
from jax.experimental.pallas import tpu as pltpu

N = 50000
E = 800000
H = 64
N_LAYERS = 6
BN = 1000
BE = 1000

_f32 = jnp.float32


def _full(shape):
    return pl.BlockSpec(shape, lambda *_: tuple(0 for _ in shape))


def _rows(shape):
    return pl.BlockSpec(shape, lambda i: (0,) * (len(shape) - 2) + (i, 0))


def _enc_body(x_ref, w1, b1, w2, b2, a0, b0, h_ref, p_ref, q_ref):
    t = jnp.maximum(jnp.dot(x_ref[...], w1[...], preferred_element_type=_f32, precision=lax.Precision.HIGHEST) + b1[...], 0.0)
    h = jnp.dot(t, w2[...], preferred_element_type=_f32, precision=lax.Precision.HIGHEST) + b2[...]
    h_ref[...] = h
    p_ref[...] = jnp.dot(h, a0[...], preferred_element_type=_f32, precision=lax.Precision.HIGHEST)
    q_ref[...] = jnp.dot(h, b0[...], preferred_element_type=_f32, precision=lax.Precision.HIGHEST)


def _enc_nodes(x, w1, b1, w2, b2, a0, b0):
    return pl.pallas_call(
        _enc_body,
        grid=(N // BN,),
        in_specs=[
            pl.BlockSpec((BN, x.shape[1]), lambda i: (i, 0)),
            _full(w1.shape), _full(b1.shape), _full(w2.shape), _full(b2.shape),
            _full(a0.shape), _full(b0.shape),
        ],
        out_specs=[_rows((BN, H)), _rows((BN, H)), _rows((BN, H))],
        out_shape=[jax.ShapeDtypeStruct((N, H), _f32)] * 3,
    )(x, w1, b1, w2, b2, a0, b0)


def _ec_body(ea_ref, w1, b1, w2, b2, cs, b1s, ec_ref):
    t = jnp.maximum(jnp.dot(ea_ref[...], w1[...], preferred_element_type=_f32, precision=lax.Precision.HIGHEST) + b1[...], 0.0)
    e = jnp.dot(t, w2[...], preferred_element_type=_f32, precision=lax.Precision.HIGHEST) + b2[...]
    for l in range(N_LAYERS):
        ec_ref[l] = jnp.dot(e, cs[l], preferred_element_type=_f32, precision=lax.Precision.HIGHEST) + b1s[l]


def _ec_all(ea, w1, b1, w2, b2, cs, b1s):
    return pl.pallas_call(
        _ec_body,
        grid=(E // BE,),
        in_specs=[
            pl.BlockSpec((BE, ea.shape[1]), lambda i: (i, 0)),
            _full(w1.shape), _full(b1.shape), _full(w2.shape), _full(b2.shape),
            pl.BlockSpec(cs.shape, lambda i: (0, 0, 0)),
            pl.BlockSpec(b1s.shape, lambda i: (0, 0, 0)),
        ],
        out_specs=pl.BlockSpec((N_LAYERS, BE, H), lambda i: (0, i, 0)),
        out_shape=jax.ShapeDtypeStruct((N_LAYERS, E, H), _f32),
    )(ea, w1, b1, w2, b2, cs, b1s)


def _upd_body(h_ref, s_ref, deg_ref, w2e, b2e, v1a, v1b, b1n, v2, b2n, an, bn,
              h_out, p_out, q_out):
    h = h_ref[...]
    agg = jnp.dot(s_ref[...], w2e[...], preferred_element_type=_f32, precision=lax.Precision.HIGHEST) + deg_ref[...] * b2e[...]
    t = jnp.maximum(
        jnp.dot(h, v1a[...], preferred_element_type=_f32, precision=lax.Precision.HIGHEST)
        + jnp.dot(agg, v1b[...], preferred_element_type=_f32, precision=lax.Precision.HIGHEST) + b1n[...], 0.0)
    hn = h + jnp.dot(t, v2[...], preferred_element_type=_f32, precision=lax.Precision.HIGHEST) + b2n[...]
    h_out[...] = hn
    p_out[...] = jnp.dot(hn, an[...], preferred_element_type=_f32, precision=lax.Precision.HIGHEST)
    q_out[...] = jnp.dot(hn, bn[...], preferred_element_type=_f32, precision=lax.Precision.HIGHEST)


def _node_update(h, s, deg, w2e, b2e, v1a, v1b, b1n, v2, b2n, an, bn):
    return pl.pallas_call(
        _upd_body,
        grid=(N // BN,),
        in_specs=[
            _rows((BN, H)), _rows((BN, H)), pl.BlockSpec((BN, 1), lambda i: (i, 0)),
            _full(w2e.shape), _full(b2e.shape), _full(v1a.shape), _full(v1b.shape),
            _full(b1n.shape), _full(v2.shape), _full(b2n.shape),
            _full(an.shape), _full(bn.shape),
        ],
        out_specs=[_rows((BN, H))] * 3,
        out_shape=[jax.ShapeDtypeStruct((N, H), _f32)] * 3,
    )(h, s, deg, w2e, b2e, v1a, v1b, b1n, v2, b2n, an, bn)


def _last_body(h_ref, s_ref, deg_ref, w2e, b2e, v1a, v1b, b1n, v2, b2n,
               d1, db1, d2, db2, d3, db3, fac_ref, out_ref):
    h = h_ref[...]
    agg = jnp.dot(s_ref[...], w2e[...], preferred_element_type=_f32, precision=lax.Precision.HIGHEST) + deg_ref[...] * b2e[...]
    t = jnp.maximum(
        jnp.dot(h, v1a[...], preferred_element_type=_f32, precision=lax.Precision.HIGHEST)
        + jnp.dot(agg, v1b[...], preferred_element_type=_f32, precision=lax.Precision.HIGHEST) + b1n[...], 0.0)
    hn = h + jnp.dot(t, v2[...], preferred_element_type=_f32, precision=lax.Precision.HIGHEST) + b2n[...]
    u = jnp.maximum(jnp.dot(hn, d1[...], preferred_element_type=_f32, precision=lax.Precision.HIGHEST) + db1[...], 0.0)
    u = jnp.maximum(jnp.dot(u, d2[...], preferred_element_type=_f32, precision=lax.Precision.HIGHEST) + db2[...], 0.0)
    raw = jnp.dot(u, d3[...], preferred_element_type=_f32, precision=lax.Precision.HIGHEST) + db3[...]
    out_ref[...] = raw * fac_ref[...]


def _last_update(h, s, deg, w2e, b2e, v1a, v1b, b1n, v2, b2n, dec_ws, fac):
    d1, db1, d2, db2, d3, db3 = dec_ws
    return pl.pallas_call(
        _last_body,
        grid=(N // BN,),
        in_specs=[
            _rows((BN, H)), _rows((BN, H)), pl.BlockSpec((BN, 1), lambda i: (i, 0)),
            _full(w2e.shape), _full(b2e.shape), _full(v1a.shape), _full(v1b.shape),
            _full(b1n.shape), _full(v2.shape), _full(b2n.shape),
            _full(d1.shape), _full(db1.shape), _full(d2.shape), _full(db2.shape),
            _full(d3.shape), _full(db3.shape),
            pl.BlockSpec((BN, 3), lambda i: (i, 0)),
        ],
        out_specs=pl.BlockSpec((BN, 3), lambda i: (i, 0)),
        out_shape=jax.ShapeDtypeStruct((N, 3), _f32),
    )(h, s, deg, w2e, b2e, v1a, v1b, b1n, v2, b2n, d1, db1, d2, db2, d3, db3, fac)


def kernel(x, edge_index, edge_attr, u_c, theta_c, bc_disp, bc_rot, params):
    src = edge_index[0]
    dst = edge_index[1]

    def r2(b):
        return b.reshape(1, -1)

    ne = params['node_enc']
    ee = params['edge_enc']
    mp = params['mp']
    dec = params['dec']

    As = [lp['edge'][0][0][0:H] for lp in mp]
    Bs = [lp['edge'][0][0][H:2 * H] for lp in mp]
    Cs = jnp.stack([lp['edge'][0][0][2 * H:3 * H] for lp in mp])
    b1s = jnp.stack([lp['edge'][0][1].reshape(1, H) for lp in mp])
    V1as = [lp['node'][0][0][0:H] for lp in mp]
    V1bs = [lp['node'][0][0][H:2 * H] for lp in mp]

    h, P, Q = _enc_nodes(x, ne[0][0], r2(ne[0][1]), ne[1][0], r2(ne[1][1]),
                         As[0], Bs[0])
    ec = _ec_all(edge_attr, ee[0][0], r2(ee[0][1]), ee[1][0], r2(ee[1][1]),
                 Cs, b1s)

    deg = jax.ops.segment_sum(jnp.ones((E,), _f32), dst, num_segments=N).reshape(N, 1)

    fac = jnp.concatenate([
        u_c.reshape(N, 1) * (1.0 - bc_disp),
        u_c.reshape(N, 1) * (1.0 - bc_disp),
        theta_c.reshape(N, 1) * (1.0 - bc_rot)], axis=1)

    for l in range(N_LAYERS):
        lp = mp[l]
        z = jnp.maximum(P[src] + Q[dst] + ec[l], 0.0)
        s = jax.ops.segment_sum(z, dst, num_segments=N)
        w2e, b2e = lp['edge'][1][0], r2(lp['edge'][1][1])
        b1n = r2(lp['node'][0][1])
        v2, b2n = lp['node'][1][0], r2(lp['node'][1][1])
        if l < N_LAYERS - 1:
            h, P, Q = _node_update(h, s, deg, w2e, b2e, V1as[l], V1bs[l], b1n,
                                   v2, b2n, As[l + 1], Bs[l + 1])
        else:
            dec_ws = (dec[0][0], r2(dec[0][1]), dec[1][0], r2(dec[1][1]),
                      dec[2][0], r2(dec[2][1]))
            out = _last_update(h, s, deg, w2e, b2e, V1as[l], V1bs[l], b1n,
                               v2, b2n, dec_ws, fac)
    return out
